# baseline (speedup 1.0000x reference)
import jax
import jax.numpy as jnp
from jax.experimental import pallas as pl
from jax.experimental.pallas import tpu as pltpu

_HIDDEN = 128
_NCOLORS = 10
_KPAD = 16
_LB = 128
_HB = 6


def _embed_block(grid_ref, tab_ref, pos_ref, out_ref):
    hb, w, lb = grid_ref.shape
    g = grid_ref[...]
    oh = (g[..., None] == jax.lax.broadcasted_iota(
        jnp.int32, (hb, w, lb, _KPAD), 3)).astype(jnp.float32)
    x = jnp.dot(oh.reshape(hb * w * lb, _KPAD), tab_ref[...],
                preferred_element_type=jnp.float32)
    out_ref[...] = x.reshape(hb, w, lb, _HIDDEN) + pos_ref[...][:, :, None, :]


def kernel(grid, color_embed, pos_embed):
    b, h, w = grid.shape
    gt = jnp.transpose(grid.astype(jnp.int32), (1, 2, 0))
    tab = jnp.zeros((_KPAD, _HIDDEN), jnp.float32).at[:_NCOLORS].set(color_embed)
    pos = pos_embed[:h, :w]
    out = pl.pallas_call(
        _embed_block,
        grid=(b // _LB, h // _HB),
        in_specs=[
            pl.BlockSpec((_HB, w, _LB), lambda i, j: (j, 0, i)),
            pl.BlockSpec((_KPAD, _HIDDEN), lambda i, j: (0, 0)),
            pl.BlockSpec((_HB, w, _HIDDEN), lambda i, j: (j, 0, 0)),
        ],
        out_specs=pl.BlockSpec((_HB, w, _LB, _HIDDEN), lambda i, j: (j, 0, i, 0)),
        out_shape=jax.ShapeDtypeStruct((h, w, b, _HIDDEN), jnp.float32),
        compiler_params=pltpu.CompilerParams(
            dimension_semantics=("parallel", "parallel")),
    )(gt, tab, pos)
    return jnp.transpose(out, (2, 0, 1, 3))


# speedup vs baseline: 1.0213x; 1.0213x over previous
import jax
import jax.numpy as jnp
from jax.experimental import pallas as pl
from jax.experimental.pallas import tpu as pltpu

_HIDDEN = 128
_NCOLORS = 10
_KPAD = 16
_LB = 1024
_HB = 1


def _embed_block(grid_ref, tab_ref, pos_ref, out_ref):
    hb, w, lb = grid_ref.shape
    g = grid_ref[...]
    oh = (g[..., None] == jax.lax.broadcasted_iota(
        jnp.int32, (hb, w, lb, _KPAD), 3)).astype(jnp.float32)
    x = jnp.dot(oh.reshape(hb * w * lb, _KPAD), tab_ref[...],
                preferred_element_type=jnp.float32)
    out_ref[...] = x.reshape(hb, w, lb, _HIDDEN) + pos_ref[...][:, :, None, :]


def kernel(grid, color_embed, pos_embed):
    b, h, w = grid.shape
    gt = jnp.transpose(grid.astype(jnp.int32), (1, 2, 0))
    tab = jnp.zeros((_KPAD, _HIDDEN), jnp.float32).at[:_NCOLORS].set(color_embed)
    pos = pos_embed[:h, :w]
    out = pl.pallas_call(
        _embed_block,
        grid=(h // _HB,),
        in_specs=[
            pl.BlockSpec((_HB, w, _LB), lambda i: (i, 0, 0)),
            pl.BlockSpec((_KPAD, _HIDDEN), lambda i: (0, 0)),
            pl.BlockSpec((_HB, w, _HIDDEN), lambda i: (i, 0, 0)),
        ],
        out_specs=pl.BlockSpec((_HB, w, _LB, _HIDDEN), lambda i: (i, 0, 0, 0)),
        out_shape=jax.ShapeDtypeStruct((h, w, b, _HIDDEN), jnp.float32),
        compiler_params=pltpu.CompilerParams(
            dimension_semantics=("parallel",)),
    )(gt, tab, pos)
    return jnp.transpose(out, (2, 0, 1, 3))
